# baseline (speedup 1.0000x reference)
import functools

import jax
import jax.numpy as jnp
from jax import lax
from jax.experimental import pallas as pl
from jax.experimental.pallas import tpu as pltpu
from jax.experimental.pallas import tpu_sc as plsc

NUM_NODES = 1000
BATCH = 5
D = 128
EDIM = 16
N_EDGES = 320000
N_LAYERS = 3
N_TOTAL = BATCH * 2 * NUM_NODES
N_ACC = 10240

SC_CORES = 2
SC_SUBCORES = 16
SC_WORKERS = SC_CORES * SC_SUBCORES
W_GATHER = 128
NE_PAD = 327680
W_PER_TILE = NE_PAD // (W_GATHER * SC_WORKERS)
EBLK = 2048
NBLK = 1000

_vmesh = plsc.VectorSubcoreMesh(core_axis_name="core", subcore_axis_name="subcore")


def _edge_pre_body(ef_ref, w_ref, b_ref, out_ref):
    out_ref[...] = (
        jnp.dot(ef_ref[...], w_ref[...], preferred_element_type=jnp.float32)
        + b_ref[...]
    )


def _edge_pre(ef_p, w1cT, b1r):
    grid = (NE_PAD // EBLK,)
    return pl.pallas_call(
        _edge_pre_body,
        grid=grid,
        in_specs=[
            pl.BlockSpec((EBLK, EDIM), lambda i: (i, 0)),
            pl.BlockSpec((EDIM, D), lambda i: (0, 0)),
            pl.BlockSpec((1, D), lambda i: (0, 0)),
        ],
        out_specs=pl.BlockSpec((EBLK, D), lambda i: (i, 0)),
        out_shape=jax.ShapeDtypeStruct((NE_PAD, D), jnp.float32),
    )(ef_p, w1cT, b1r)


def _node_pre_body(ns_ref, wa_ref, wb_ref, p_ref, q_ref):
    ns = ns_ref[...]
    p_ref[...] = jnp.dot(ns, wa_ref[...], preferred_element_type=jnp.float32)
    q_ref[...] = jnp.dot(ns, wb_ref[...], preferred_element_type=jnp.float32)


def _node_pre(ns, w1aT, w1bT):
    grid = (N_TOTAL // NBLK,)
    return pl.pallas_call(
        _node_pre_body,
        grid=grid,
        in_specs=[
            pl.BlockSpec((NBLK, D), lambda i: (i, 0)),
            pl.BlockSpec((D, D), lambda i: (0, 0)),
            pl.BlockSpec((D, D), lambda i: (0, 0)),
        ],
        out_specs=[
            pl.BlockSpec((NBLK, D), lambda i: (i, 0)),
            pl.BlockSpec((NBLK, D), lambda i: (i, 0)),
        ],
        out_shape=[
            jax.ShapeDtypeStruct((N_TOTAL, D), jnp.float32),
            jax.ShapeDtypeStruct((N_TOTAL, D), jnp.float32),
        ],
    )(ns, w1aT, w1bT)


def _edge_mlp_body(off, g1_ref, g2_ref, ew_ref, w2_ref, b2_ref, w3_ref, b3_ref,
                   out_ref):
    i = pl.program_id(0)
    x1 = jnp.maximum(g1_ref[...] + g2_ref[...] + ew_ref[...], 0.0)
    x2 = jnp.maximum(
        jnp.dot(x1.astype(jnp.bfloat16), w2_ref[...],
                preferred_element_type=jnp.float32) + b2_ref[...],
        0.0,
    )
    m = jnp.dot(x2.astype(jnp.bfloat16), w3_ref[...],
                preferred_element_type=jnp.float32) + b3_ref[...]
    row = off + i * EBLK + lax.broadcasted_iota(jnp.int32, (EBLK, 1), 0)
    out_ref[...] = jnp.where(row < N_EDGES, m, 0.0)


def _edge_mlp(off, g1, g2, ew, w2T, b2r, w3T, b3r):
    grid = (NE_PAD // EBLK,)
    return pl.pallas_call(
        functools.partial(_edge_mlp_body, off),
        grid=grid,
        in_specs=[
            pl.BlockSpec((EBLK, D), lambda i: (i, 0)),
            pl.BlockSpec((EBLK, D), lambda i: (i, 0)),
            pl.BlockSpec((EBLK, D), lambda i: (i, 0)),
            pl.BlockSpec((D, D), lambda i: (0, 0)),
            pl.BlockSpec((1, D), lambda i: (0, 0)),
            pl.BlockSpec((D, D), lambda i: (0, 0)),
            pl.BlockSpec((1, D), lambda i: (0, 0)),
        ],
        out_specs=pl.BlockSpec((EBLK, D), lambda i: (i, 0)),
        out_shape=jax.ShapeDtypeStruct((NE_PAD, D), jnp.float32),
    )(g1, g2, ew, w2T, b2r, w3T, b3r)


def _update_body(ns_ref, nsp_ref, s0_ref, s1_ref,
                 wna_ref, wnb_ref, wnc_ref,
                 b1_ref, w2_ref, b2_ref, w3_ref, b3_ref, out_ref):
    ns = ns_ref[...]
    att = ns - nsp_ref[...]
    summed = s0_ref[0] + s1_ref[0]
    h = (
        jnp.dot(ns, wna_ref[...], preferred_element_type=jnp.float32)
        + jnp.dot(summed, wnb_ref[...], preferred_element_type=jnp.float32)
        + jnp.dot(att, wnc_ref[...], preferred_element_type=jnp.float32)
        + b1_ref[...]
    )
    h = jnp.maximum(h, 0.0)
    h = jnp.maximum(
        jnp.dot(h, w2_ref[...], preferred_element_type=jnp.float32) + b2_ref[...],
        0.0,
    )
    out_ref[...] = (
        jnp.dot(h, w3_ref[...], preferred_element_type=jnp.float32) + b3_ref[...]
    )


def _update(ns, partials, wnaT, wnbT, wncT, b1r, w2T, b2r, w3T, b3r):
    grid = (N_TOTAL // NBLK,)
    wspec = pl.BlockSpec((D, D), lambda i: (0, 0))
    bspec = pl.BlockSpec((1, D), lambda i: (0, 0))
    pspec0 = pl.BlockSpec((1, NBLK, D), lambda i: (0, i, 0))
    pspec1 = pl.BlockSpec((1, NBLK, D), lambda i: (1, i, 0))
    return pl.pallas_call(
        _update_body,
        grid=grid,
        in_specs=[
            pl.BlockSpec((NBLK, D), lambda i: (i, 0)),
            pl.BlockSpec((NBLK, D), lambda i: (i ^ 1, 0)),
            pspec0, pspec1,
            wspec, wspec, wspec, bspec, wspec, bspec, wspec, bspec,
        ],
        out_specs=pl.BlockSpec((NBLK, D), lambda i: (i, 0)),
        out_shape=jax.ShapeDtypeStruct((N_TOTAL, D), jnp.float32),
    )(ns, ns, partials, partials,
      wnaT, wnbT, wncT, b1r, w2T, b2r, w3T, b3r)


def _readout1_body(ns_ref, gw_ref, gb_ref, out_ref):
    ns = ns_ref[...]
    gates = jax.nn.sigmoid(
        jnp.dot(ns, gw_ref[...], preferred_element_type=jnp.float32) + gb_ref[...]
    )
    out_ref[0] = jnp.sum(ns * gates, axis=0, keepdims=True)


def _readout1(ns, gwT, gbr):
    ngroups = N_TOTAL // NUM_NODES
    return pl.pallas_call(
        _readout1_body,
        grid=(ngroups,),
        in_specs=[
            pl.BlockSpec((NUM_NODES, D), lambda i: (i, 0)),
            pl.BlockSpec((D, D), lambda i: (0, 0)),
            pl.BlockSpec((1, D), lambda i: (0, 0)),
        ],
        out_specs=pl.BlockSpec((1, 1, D), lambda i: (i, 0, 0)),
        out_shape=jax.ShapeDtypeStruct((ngroups, 1, D), jnp.float32),
    )(ns, gwT, gbr)


def _readout2_body(gs_ref, w1_ref, b1_ref, w2_ref, b2_ref, out_ref):
    g = jnp.maximum(
        jnp.dot(gs_ref[...], w1_ref[...], preferred_element_type=jnp.float32)
        + b1_ref[...],
        0.0,
    )
    out_ref[...] = (
        jnp.dot(g, w2_ref[...], preferred_element_type=jnp.float32) + b2_ref[...]
    )


def _readout2(gs, a1T, b1r, a2T, b2r):
    ngroups = gs.shape[0]
    return pl.pallas_call(
        _readout2_body,
        in_specs=[
            pl.BlockSpec((ngroups, D), lambda: (0, 0)),
            pl.BlockSpec((D, D), lambda: (0, 0)),
            pl.BlockSpec((1, D), lambda: (0, 0)),
            pl.BlockSpec((D, D), lambda: (0, 0)),
            pl.BlockSpec((1, D), lambda: (0, 0)),
        ],
        out_specs=pl.BlockSpec((ngroups, D), lambda: (0, 0)),
        out_shape=jax.ShapeDtypeStruct((ngroups, D), jnp.float32),
    )(gs, a1T, b1r, a2T, b2r)


def _sc_gather(p, q, vi2d, vj2d):
    @functools.partial(
        pl.kernel,
        out_type=[
            jax.ShapeDtypeStruct((NE_PAD, D), jnp.float32),
            jax.ShapeDtypeStruct((NE_PAD, D), jnp.float32),
        ],
        mesh=_vmesh,
        scratch_types=[
            pltpu.VMEM((W_PER_TILE, W_GATHER), jnp.int32),
            pltpu.VMEM((W_PER_TILE, W_GATHER), jnp.int32),
            pltpu.VMEM((W_GATHER, D), jnp.float32),
            pltpu.VMEM((W_GATHER, D), jnp.float32),
            pltpu.VMEM((W_GATHER, D), jnp.float32),
            pltpu.VMEM((W_GATHER, D), jnp.float32),
            pltpu.SemaphoreType.DMA,
            pltpu.SemaphoreType.DMA,
            pltpu.SemaphoreType.DMA,
            pltpu.SemaphoreType.DMA,
        ],
    )
    def k(p_hbm, q_hbm, vi_hbm, vj_hbm, g1_hbm, g2_hbm,
          vi_v, vj_v, d1_0, d2_0, d1_1, d2_1, gsem0, gsem1, wsem0, wsem1):
        cid = lax.axis_index("core")
        sid = lax.axis_index("subcore")
        wid = sid * SC_CORES + cid
        base = wid * W_PER_TILE
        pltpu.sync_copy(vi_hbm.at[pl.ds(base, W_PER_TILE)], vi_v)
        pltpu.sync_copy(vj_hbm.at[pl.ds(base, W_PER_TILE)], vj_v)

        slots = ((d1_0, d2_0, gsem0, wsem0), (d1_1, d2_1, gsem1, wsem1))

        def drain(sem, buf):
            pltpu.make_async_copy(g1_hbm.at[pl.ds(0, W_GATHER)], buf, sem).wait()

        @pl.loop(0, W_PER_TILE // 2)
        def _(t):
            for pslot in range(2):
                d1, d2, gsem, wsem = slots[pslot]
                w = 2 * t + pslot

                @pl.when(t > 0)
                def _():
                    drain(wsem, d1)
                    drain(wsem, d2)

                pltpu.async_copy(p_hbm.at[vi_v.at[w]], d1, gsem)
                pltpu.async_copy(q_hbm.at[vj_v.at[w]], d2, gsem)
            for pslot in range(2):
                d1, d2, gsem, wsem = slots[pslot]
                w = 2 * t + pslot
                row0 = (base + w) * W_GATHER
                drain(gsem, d1)
                drain(gsem, d2)
                pltpu.async_copy(d1, g1_hbm.at[pl.ds(row0, W_GATHER)], wsem)
                pltpu.async_copy(d2, g2_hbm.at[pl.ds(row0, W_GATHER)], wsem)

        for pslot in range(2):
            d1, d2, _, wsem = slots[pslot]
            drain(wsem, d1)
            drain(wsem, d2)

    return k(p, q, vi2d, vj2d)


def _sc_scatter(m, vi_c, vj_c, zeros_hbm):
    rows_per_sub = N_ACC // SC_SUBCORES

    @functools.partial(
        pl.kernel,
        out_type=jax.ShapeDtypeStruct((SC_CORES, N_ACC, D), jnp.float32),
        mesh=_vmesh,
        scratch_types=[
            pltpu.VMEM_SHARED((N_ACC, D), jnp.float32),
            pltpu.SemaphoreType.DMA,
        ],
    )
    def k(m_hbm, vi_hbm, vj_hbm, z_hbm, out_hbm, acc, sem):
        cid = lax.axis_index("core")
        sid = lax.axis_index("subcore")
        sl = pl.ds(sid * rows_per_sub, rows_per_sub)
        pltpu.sync_copy(z_hbm.at[sl], acc.at[sl])
        plsc.subcore_barrier()

        def body(m_vmem, vi_vmem, vj_vmem):
            c1 = pltpu.async_copy(m_vmem, acc.at[vi_vmem.at[0]], sem, add=True)
            c2 = pltpu.async_copy(m_vmem, acc.at[vj_vmem.at[0]], sem, add=True)
            c1.wait()
            c2.wait()

        pltpu.emit_pipeline(
            body,
            grid=(NE_PAD // W_GATHER,),
            in_specs=[
                pl.BlockSpec((W_GATHER, D), lambda i: (i, 0)),
                pl.BlockSpec((1, W_GATHER), lambda i: (i, 0)),
                pl.BlockSpec((1, W_GATHER), lambda i: (i, 0)),
            ],
            out_specs=[],
            core_axis_name=("core", "subcore"),
            dimension_semantics=(pltpu.PARALLEL,),
        )(m_hbm, vi_hbm, vj_hbm)

        plsc.subcore_barrier()
        pltpu.sync_copy(acc.at[sl], out_hbm.at[cid, sl])

    return k(m, vi_c, vj_c, zeros_hbm)


def kernel(node_features, edge_features, msg_w1, msg_b1, msg_w2, msg_b2,
           msg_w3, msg_b3, upd_w1, upd_b1, upd_w2, upd_b2, upd_w3, upd_b3,
           gate_w, gate_b, agg_w1, agg_b1, agg_w2, agg_b2, edge_vertices):
    ns = node_features.reshape(-1, D)

    pad = NE_PAD - N_EDGES
    vi_p = jnp.pad(edge_vertices[:, 0], (0, pad)).reshape(-1, W_GATHER)
    vj_p = jnp.pad(edge_vertices[:, 1], (0, pad)).reshape(-1, W_GATHER)
    ef_p = jnp.pad(edge_features, ((0, pad), (0, 0)))
    zeros_hbm = jnp.zeros((N_ACC, D), jnp.float32)

    w1aT = msg_w1[:, :D].T
    w1bT = msg_w1[:, D:2 * D].T
    w1cT = msg_w1[:, 2 * D:].T
    b1r = msg_b1.reshape(1, D)
    w2T = msg_w2.T.astype(jnp.bfloat16)
    b2r = msg_b2.reshape(1, D)
    w3T = msg_w3.T.astype(jnp.bfloat16)
    b3r = msg_b3.reshape(1, D)
    uw1T = upd_w1.T
    unaT = uw1T[:D]
    unbT = uw1T[D:D + D]
    uncT = uw1T[D + D:]
    ub1r = upd_b1.reshape(1, D)
    uw2T = upd_w2.T
    ub2r = upd_b2.reshape(1, D)
    uw3T = upd_w3.T
    ub3r = upd_b3.reshape(1, D)
    gwT = gate_w.T
    gbr = gate_b.reshape(1, D)
    a1T = agg_w1.T
    a1br = agg_b1.reshape(1, D)
    a2T = agg_w2.T
    a2br = agg_b2.reshape(1, D)

    ew = _edge_pre(ef_p, w1cT, b1r)

    for _ in range(N_LAYERS):
        p, q = _node_pre(ns, w1aT, w1bT)
        g1, g2 = _sc_gather(p, q, vi_p, vj_p)
        m = _edge_mlp(0, g1, g2, ew, w2T, b2r, w3T, b3r)
        partials = _sc_scatter(m, vi_p, vj_p, zeros_hbm)
        ns = _update(ns, partials,
                     unaT, unbT, uncT, ub1r, uw2T, ub2r, uw3T, ub3r)

    gs = _readout1(ns, gwT, gbr)
    out = _readout2(gs.reshape(-1, D), a1T, a1br, a2T, a2br)
    return out.reshape(BATCH, 2, D)


# speedup vs baseline: 1.5905x; 1.5905x over previous
import functools

import jax
import jax.numpy as jnp
from jax import lax
from jax.experimental import pallas as pl
from jax.experimental.pallas import tpu as pltpu
from jax.experimental.pallas import tpu_sc as plsc

NUM_NODES = 1000
BATCH = 5
D = 128
EDIM = 16
N_EDGES = 320000
N_LAYERS = 3
N_TOTAL = BATCH * 2 * NUM_NODES
N_ACC = 10240

SC_CORES = 2
SC_SUBCORES = 16
SC_WORKERS = SC_CORES * SC_SUBCORES
W_GATHER = 128
NE_PAD = 323584
EBLK = 2048
NBLK = 1000

_vmesh = plsc.VectorSubcoreMesh(core_axis_name="core", subcore_axis_name="subcore")


def _edge_pre_body(ef_ref, w_ref, b_ref, out_ref):
    out_ref[...] = (
        jnp.dot(ef_ref[...], w_ref[...], preferred_element_type=jnp.float32)
        + b_ref[...]
    ).astype(jnp.bfloat16)


def _edge_pre(ef_p, w1cT, b1r):
    grid = (NE_PAD // EBLK,)
    return pl.pallas_call(
        _edge_pre_body,
        grid=grid,
        in_specs=[
            pl.BlockSpec((EBLK, EDIM), lambda i: (i, 0)),
            pl.BlockSpec((EDIM, D), lambda i: (0, 0)),
            pl.BlockSpec((1, D), lambda i: (0, 0)),
        ],
        out_specs=pl.BlockSpec((EBLK, D), lambda i: (i, 0)),
        out_shape=jax.ShapeDtypeStruct((NE_PAD, D), jnp.bfloat16),
    )(ef_p, w1cT, b1r)


def _node_pre_body(ns_ref, wa_ref, wb_ref, p_ref, q_ref):
    ns = ns_ref[...]
    p_ref[...] = jnp.dot(ns, wa_ref[...], preferred_element_type=jnp.float32)
    q_ref[...] = jnp.dot(ns, wb_ref[...], preferred_element_type=jnp.float32)


def _node_pre(ns, w1aT, w1bT):
    grid = (N_TOTAL // NBLK,)
    return pl.pallas_call(
        _node_pre_body,
        grid=grid,
        in_specs=[
            pl.BlockSpec((NBLK, D), lambda i: (i, 0)),
            pl.BlockSpec((D, D), lambda i: (0, 0)),
            pl.BlockSpec((D, D), lambda i: (0, 0)),
        ],
        out_specs=[
            pl.BlockSpec((NBLK, D), lambda i: (i, 0)),
            pl.BlockSpec((NBLK, D), lambda i: (i, 0)),
        ],
        out_shape=[
            jax.ShapeDtypeStruct((N_TOTAL, D), jnp.float32),
            jax.ShapeDtypeStruct((N_TOTAL, D), jnp.float32),
        ],
    )(ns, w1aT, w1bT)


def _edge_mlp_body(off, g1_ref, g2_ref, ew_ref, w2_ref, b2_ref, w3_ref, b3_ref,
                   out_ref):
    i = pl.program_id(0)
    x1 = jnp.maximum(
        g1_ref[...] + g2_ref[...] + ew_ref[...].astype(jnp.float32), 0.0)
    x2 = jnp.maximum(
        jnp.dot(x1.astype(jnp.bfloat16), w2_ref[...],
                preferred_element_type=jnp.float32) + b2_ref[...],
        0.0,
    )
    m = jnp.dot(x2.astype(jnp.bfloat16), w3_ref[...],
                preferred_element_type=jnp.float32) + b3_ref[...]
    row = off + i * EBLK + lax.broadcasted_iota(jnp.int32, (EBLK, 1), 0)
    out_ref[...] = jnp.where(row < N_EDGES, m, 0.0)


def _edge_mlp(off, g1, g2, ew, w2T, b2r, w3T, b3r):
    grid = (NE_PAD // EBLK,)
    return pl.pallas_call(
        functools.partial(_edge_mlp_body, off),
        grid=grid,
        in_specs=[
            pl.BlockSpec((EBLK, D), lambda i: (i, 0)),
            pl.BlockSpec((EBLK, D), lambda i: (i, 0)),
            pl.BlockSpec((EBLK, D), lambda i: (i, 0)),
            pl.BlockSpec((D, D), lambda i: (0, 0)),
            pl.BlockSpec((1, D), lambda i: (0, 0)),
            pl.BlockSpec((D, D), lambda i: (0, 0)),
            pl.BlockSpec((1, D), lambda i: (0, 0)),
        ],
        out_specs=pl.BlockSpec((EBLK, D), lambda i: (i, 0)),
        out_shape=jax.ShapeDtypeStruct((NE_PAD, D), jnp.float32),
    )(g1, g2, ew, w2T, b2r, w3T, b3r)


def _update_body(ns_ref, nsp_ref, s0_ref, s1_ref,
                 wna_ref, wnb_ref, wnc_ref,
                 b1_ref, w2_ref, b2_ref, w3_ref, b3_ref, out_ref):
    ns = ns_ref[...]
    att = ns - nsp_ref[...]
    summed = s0_ref[0] + s1_ref[0]
    h = (
        jnp.dot(ns, wna_ref[...], preferred_element_type=jnp.float32)
        + jnp.dot(summed, wnb_ref[...], preferred_element_type=jnp.float32)
        + jnp.dot(att, wnc_ref[...], preferred_element_type=jnp.float32)
        + b1_ref[...]
    )
    h = jnp.maximum(h, 0.0)
    h = jnp.maximum(
        jnp.dot(h, w2_ref[...], preferred_element_type=jnp.float32) + b2_ref[...],
        0.0,
    )
    out_ref[...] = (
        jnp.dot(h, w3_ref[...], preferred_element_type=jnp.float32) + b3_ref[...]
    )


def _update(ns, partials, wnaT, wnbT, wncT, b1r, w2T, b2r, w3T, b3r):
    grid = (N_TOTAL // NBLK,)
    wspec = pl.BlockSpec((D, D), lambda i: (0, 0))
    bspec = pl.BlockSpec((1, D), lambda i: (0, 0))
    pspec0 = pl.BlockSpec((1, NBLK, D), lambda i: (0, i, 0))
    pspec1 = pl.BlockSpec((1, NBLK, D), lambda i: (1, i, 0))
    return pl.pallas_call(
        _update_body,
        grid=grid,
        in_specs=[
            pl.BlockSpec((NBLK, D), lambda i: (i, 0)),
            pl.BlockSpec((NBLK, D), lambda i: (i ^ 1, 0)),
            pspec0, pspec1,
            wspec, wspec, wspec, bspec, wspec, bspec, wspec, bspec,
        ],
        out_specs=pl.BlockSpec((NBLK, D), lambda i: (i, 0)),
        out_shape=jax.ShapeDtypeStruct((N_TOTAL, D), jnp.float32),
    )(ns, ns, partials, partials,
      wnaT, wnbT, wncT, b1r, w2T, b2r, w3T, b3r)


def _readout1_body(ns_ref, gw_ref, gb_ref, out_ref):
    ns = ns_ref[...]
    gates = jax.nn.sigmoid(
        jnp.dot(ns, gw_ref[...], preferred_element_type=jnp.float32) + gb_ref[...]
    )
    out_ref[0] = jnp.sum(ns * gates, axis=0, keepdims=True)


def _readout1(ns, gwT, gbr):
    ngroups = N_TOTAL // NUM_NODES
    return pl.pallas_call(
        _readout1_body,
        grid=(ngroups,),
        in_specs=[
            pl.BlockSpec((NUM_NODES, D), lambda i: (i, 0)),
            pl.BlockSpec((D, D), lambda i: (0, 0)),
            pl.BlockSpec((1, D), lambda i: (0, 0)),
        ],
        out_specs=pl.BlockSpec((1, 1, D), lambda i: (i, 0, 0)),
        out_shape=jax.ShapeDtypeStruct((ngroups, 1, D), jnp.float32),
    )(ns, gwT, gbr)


def _readout2_body(gs_ref, w1_ref, b1_ref, w2_ref, b2_ref, out_ref):
    g = jnp.maximum(
        jnp.dot(gs_ref[...], w1_ref[...], preferred_element_type=jnp.float32)
        + b1_ref[...],
        0.0,
    )
    out_ref[...] = (
        jnp.dot(g, w2_ref[...], preferred_element_type=jnp.float32) + b2_ref[...]
    )


def _readout2(gs, a1T, b1r, a2T, b2r):
    ngroups = gs.shape[0]
    return pl.pallas_call(
        _readout2_body,
        in_specs=[
            pl.BlockSpec((ngroups, D), lambda: (0, 0)),
            pl.BlockSpec((D, D), lambda: (0, 0)),
            pl.BlockSpec((1, D), lambda: (0, 0)),
            pl.BlockSpec((D, D), lambda: (0, 0)),
            pl.BlockSpec((1, D), lambda: (0, 0)),
        ],
        out_specs=pl.BlockSpec((ngroups, D), lambda: (0, 0)),
        out_shape=jax.ShapeDtypeStruct((ngroups, D), jnp.float32),
    )(gs, a1T, b1r, a2T, b2r)


def _sc_gather(p, q, vi2d, vj2d):
    @functools.partial(
        pl.kernel,
        out_type=[
            jax.ShapeDtypeStruct((NE_PAD, D), jnp.float32),
            jax.ShapeDtypeStruct((NE_PAD, D), jnp.float32),
        ],
        mesh=_vmesh,
        scratch_types=[pltpu.SemaphoreType.DMA],
    )
    def k(p_hbm, q_hbm, vi_hbm, vj_hbm, g1_hbm, g2_hbm, sem):
        def body(vi_vmem, vj_vmem, g1_vmem, g2_vmem):
            c1 = pltpu.async_copy(p_hbm.at[vi_vmem.at[0]], g1_vmem, sem)
            c2 = pltpu.async_copy(q_hbm.at[vj_vmem.at[0]], g2_vmem, sem)
            c1.wait()
            c2.wait()

        pltpu.emit_pipeline(
            body,
            grid=(NE_PAD // W_GATHER,),
            in_specs=[
                pl.BlockSpec((1, W_GATHER), lambda i: (i, 0)),
                pl.BlockSpec((1, W_GATHER), lambda i: (i, 0)),
            ],
            out_specs=[
                pl.BlockSpec((W_GATHER, D), lambda i: (i, 0)),
                pl.BlockSpec((W_GATHER, D), lambda i: (i, 0)),
            ],
            core_axis_name=("core", "subcore"),
            dimension_semantics=(pltpu.PARALLEL,),
        )(vi_hbm, vj_hbm, g1_hbm, g2_hbm)

    return k(p, q, vi2d, vj2d)


def _sc_scatter(m, vi_c, vj_c, zeros_hbm):
    rows_per_sub = N_ACC // SC_SUBCORES

    @functools.partial(
        pl.kernel,
        out_type=jax.ShapeDtypeStruct((SC_CORES, N_ACC, D), jnp.float32),
        mesh=_vmesh,
        scratch_types=[
            pltpu.VMEM_SHARED((N_ACC, D), jnp.float32),
            pltpu.SemaphoreType.DMA,
        ],
    )
    def k(m_hbm, vi_hbm, vj_hbm, z_hbm, out_hbm, acc, sem):
        cid = lax.axis_index("core")
        sid = lax.axis_index("subcore")
        sl = pl.ds(sid * rows_per_sub, rows_per_sub)
        pltpu.sync_copy(z_hbm.at[sl], acc.at[sl])
        plsc.subcore_barrier()

        def body(m_vmem, vi_vmem, vj_vmem):
            c1 = pltpu.async_copy(m_vmem, acc.at[vi_vmem.at[0]], sem, add=True)
            c2 = pltpu.async_copy(m_vmem, acc.at[vj_vmem.at[0]], sem, add=True)
            c1.wait()
            c2.wait()

        pltpu.emit_pipeline(
            body,
            grid=(NE_PAD // W_GATHER,),
            in_specs=[
                pl.BlockSpec((W_GATHER, D), lambda i: (i, 0)),
                pl.BlockSpec((1, W_GATHER), lambda i: (i, 0)),
                pl.BlockSpec((1, W_GATHER), lambda i: (i, 0)),
            ],
            out_specs=[],
            core_axis_name=("core", "subcore"),
            dimension_semantics=(pltpu.PARALLEL,),
        )(m_hbm, vi_hbm, vj_hbm)

        plsc.subcore_barrier()
        pltpu.sync_copy(acc.at[sl], out_hbm.at[cid, sl])

    return k(m, vi_c, vj_c, zeros_hbm)


def kernel(node_features, edge_features, msg_w1, msg_b1, msg_w2, msg_b2,
           msg_w3, msg_b3, upd_w1, upd_b1, upd_w2, upd_b2, upd_w3, upd_b3,
           gate_w, gate_b, agg_w1, agg_b1, agg_w2, agg_b2, edge_vertices):
    ns = node_features.reshape(-1, D)

    pad = NE_PAD - N_EDGES
    vi_p = jnp.pad(edge_vertices[:, 0], (0, pad)).reshape(-1, W_GATHER)
    vj_p = jnp.pad(edge_vertices[:, 1], (0, pad)).reshape(-1, W_GATHER)
    ef_p = jnp.pad(edge_features, ((0, pad), (0, 0)))
    zeros_hbm = jnp.zeros((N_ACC, D), jnp.float32)

    w1aT = msg_w1[:, :D].T
    w1bT = msg_w1[:, D:2 * D].T
    w1cT = msg_w1[:, 2 * D:].T
    b1r = msg_b1.reshape(1, D)
    w2T = msg_w2.T.astype(jnp.bfloat16)
    b2r = msg_b2.reshape(1, D)
    w3T = msg_w3.T.astype(jnp.bfloat16)
    b3r = msg_b3.reshape(1, D)
    uw1T = upd_w1.T
    unaT = uw1T[:D]
    unbT = uw1T[D:D + D]
    uncT = uw1T[D + D:]
    ub1r = upd_b1.reshape(1, D)
    uw2T = upd_w2.T
    ub2r = upd_b2.reshape(1, D)
    uw3T = upd_w3.T
    ub3r = upd_b3.reshape(1, D)
    gwT = gate_w.T
    gbr = gate_b.reshape(1, D)
    a1T = agg_w1.T
    a1br = agg_b1.reshape(1, D)
    a2T = agg_w2.T
    a2br = agg_b2.reshape(1, D)

    ew = _edge_pre(ef_p, w1cT, b1r)

    for _ in range(N_LAYERS):
        p, q = _node_pre(ns, w1aT, w1bT)
        g1, g2 = _sc_gather(p, q, vi_p, vj_p)
        m = _edge_mlp(0, g1, g2, ew, w2T, b2r, w3T, b3r)
        partials = _sc_scatter(m, vi_p, vj_p, zeros_hbm)
        ns = _update(ns, partials,
                     unaT, unbT, uncT, ub1r, uw2T, ub2r, uw3T, ub3r)

    gs = _readout1(ns, gwT, gbr)
    out = _readout2(gs.reshape(-1, D), a1T, a1br, a2T, a2br)
    return out.reshape(BATCH, 2, D)
